# Initial kernel scaffold; baseline (speedup 1.0000x reference)
import jax
import jax.numpy as jnp
from jax.experimental import pallas as pl


def kernel(features, nodes, neighbors, W, b, a_w, a_b):
    raise NotImplementedError("write your pallas kernel here")


# speedup vs baseline: 2.0736x; 2.0736x over previous
import functools

import jax
import jax.numpy as jnp
from jax import lax
from jax.experimental import pallas as pl
from jax.experimental.pallas import tpu as pltpu
from jax.experimental.pallas import tpu_sc as plsc

N = 10000
DEG = 16
DIN = 256
DOUT = 256
L = 16
NW = 32
G = 8
BLKS = 40
PW = BLKS * G
NPAD = NW * PW


def _tc_body(x_ref, w_ref, b_ref, a_ref, c_ref, h_ref, s2_ref):
    h = jnp.dot(x_ref[...], w_ref[...], preferred_element_type=jnp.float32)
    h = h + b_ref[...]
    h_ref[...] = h
    s2_ref[...] = jnp.dot(h, a_ref[...], preferred_element_type=jnp.float32) + c_ref[...]


def _tc_stage(x, W, b, A, c):
    nb = 10
    rows = N // nb
    return pl.pallas_call(
        _tc_body,
        grid=(nb,),
        in_specs=[
            pl.BlockSpec((rows, DIN), lambda i: (i, 0)),
            pl.BlockSpec((DIN, DOUT), lambda i: (0, 0)),
            pl.BlockSpec((1, DOUT), lambda i: (0, 0)),
            pl.BlockSpec((DOUT, 128), lambda i: (0, 0)),
            pl.BlockSpec((1, 128), lambda i: (0, 0)),
        ],
        out_specs=[
            pl.BlockSpec((rows, DOUT), lambda i: (i, 0)),
            pl.BlockSpec((rows, 128), lambda i: (i, 0)),
        ],
        out_shape=[
            jax.ShapeDtypeStruct((N, DOUT), jnp.float32),
            jax.ShapeDtypeStruct((N, 128), jnp.float32),
        ],
    )(x, W, b, A, c)


def _sc_node(g, blk, nbrs_ref, sself_ref, snbr_ref, rows_ref, out_ref):
    idx = nbrs_ref[blk, pl.ds(g * L, L)]
    sg = plsc.load_gather(snbr_ref, [idx])
    su = sself_ref[pl.ds(blk * G + g, L)][0]
    x = sg + su
    e = jnp.where(x >= 0.0, x, x * jnp.float32(0.01))
    m = jnp.max(e)
    ex = jnp.exp(e - m)
    z = jnp.sum(ex)
    alpha = ex / lax.broadcast_in_dim(z, (L,), ())
    acc = [jnp.zeros((L,), jnp.float32) for _ in range(DOUT // L)]
    for k in range(DEG):
        ak = alpha[k]
        row = g * DEG + k
        for j in range(DOUT // L):
            acc[j] = acc[j] + ak * rows_ref[row, pl.ds(j * L, L)]
    for j in range(DOUT // L):
        out_ref[g, pl.ds(j * L, L)] = acc[j]


def _sc_body(h_hbm, snbr_hbm, sself_hbm, nbrs_hbm, out_hbm,
             snbr_v, sself_v, nbrs_v, rows_v, out_v, gsem, osem):
    wid = lax.axis_index("s") * 2 + lax.axis_index("c")
    base = wid * PW
    pltpu.sync_copy(snbr_hbm, snbr_v)
    pltpu.sync_copy(sself_hbm.at[wid], sself_v)
    pltpu.sync_copy(nbrs_hbm.at[wid], nbrs_v)

    def blk_body(blk, carry):
        pltpu.async_copy(h_hbm.at[nbrs_v.at[blk]], rows_v, gsem).wait()

        def g_body(g, c2):
            _sc_node(g, blk, nbrs_v, sself_v, snbr_v, rows_v, out_v)
            return c2

        lax.fori_loop(0, G, g_body, 0)
        pltpu.async_copy(out_v, out_hbm.at[pl.ds(base + blk * G, G)], osem).wait()
        return carry

    lax.fori_loop(0, BLKS, blk_body, 0)


def _sc_stage(h, s_nbr, sself_w, nbrs_w):
    mesh = plsc.VectorSubcoreMesh(core_axis_name="c", subcore_axis_name="s")
    fn = pl.kernel(
        _sc_body,
        out_type=jax.ShapeDtypeStruct((NPAD, DOUT), jnp.float32),
        mesh=mesh,
        compiler_params=pltpu.CompilerParams(needs_layout_passes=False),
        scratch_types=[
            pltpu.VMEM((N,), jnp.float32),
            pltpu.VMEM((PW + L,), jnp.float32),
            pltpu.VMEM((BLKS, G * DEG), jnp.int32),
            pltpu.VMEM((G * DEG, DOUT), jnp.float32),
            pltpu.VMEM((G, DOUT), jnp.float32),
            pltpu.SemaphoreType.DMA,
            pltpu.SemaphoreType.DMA,
        ],
    )
    return fn(h, s_nbr, sself_w, nbrs_w)


@jax.jit
def _run(features, neighbors, W, b, a_w, a_b):
    A = jnp.zeros((DOUT, 128), jnp.float32)
    A = A.at[:, 0].set(a_w[:DOUT]).at[:, 1].set(a_w[DOUT:])
    c = jnp.zeros((1, 128), jnp.float32).at[0, 0].set(a_b)
    h, s2 = _tc_stage(features, W, b.reshape(1, DOUT), A, c)
    s_self = s2[:, 0]
    s_nbr = s2[:, 1]
    sself_w = jnp.pad(jnp.pad(s_self, (0, NPAD - N)).reshape(NW, PW), ((0, 0), (0, L)))
    nbrs_w = jnp.pad(neighbors, ((0, NPAD - N), (0, 0))).reshape(NW, BLKS, G * DEG)
    out = _sc_stage(h, s_nbr, sself_w, nbrs_w)
    return out[:N]


def kernel(features, nodes, neighbors, W, b, a_w, a_b):
    del nodes
    return _run(features, neighbors, W, b, a_w, a_b)


# speedup vs baseline: 2.5122x; 1.2116x over previous
import functools

import jax
import jax.numpy as jnp
from jax import lax
from jax.experimental import pallas as pl
from jax.experimental.pallas import tpu as pltpu
from jax.experimental.pallas import tpu_sc as plsc

N = 10000
DEG = 16
DIN = 256
DOUT = 256
L = 16
NW = 32
G = 8
BLKS = 40
PW = BLKS * G
NPAD = NW * PW


def _tc_body(x_ref, w_ref, b_ref, a_ref, c_ref, h_ref, s2_ref):
    h = jnp.dot(x_ref[...], w_ref[...], preferred_element_type=jnp.float32)
    h = h + b_ref[...]
    h_ref[...] = h
    s2_ref[...] = jnp.dot(h, a_ref[...], preferred_element_type=jnp.float32) + c_ref[...]


def _tc_stage(x, W, b, A, c):
    nb = 10
    rows = N // nb
    return pl.pallas_call(
        _tc_body,
        grid=(nb,),
        in_specs=[
            pl.BlockSpec((rows, DIN), lambda i: (i, 0)),
            pl.BlockSpec((DIN, DOUT), lambda i: (0, 0)),
            pl.BlockSpec((1, DOUT), lambda i: (0, 0)),
            pl.BlockSpec((DOUT, 128), lambda i: (0, 0)),
            pl.BlockSpec((1, 128), lambda i: (0, 0)),
        ],
        out_specs=[
            pl.BlockSpec((rows, DOUT), lambda i: (i, 0)),
            pl.BlockSpec((rows, 128), lambda i: (i, 0)),
        ],
        out_shape=[
            jax.ShapeDtypeStruct((N, DOUT), jnp.float32),
            jax.ShapeDtypeStruct((N, 128), jnp.float32),
        ],
    )(x, W, b, A, c)


def _sc_node(g, blk, nbrs_ref, sself_ref, snbr_ref, rows_ref, out_ref):
    idx = nbrs_ref[blk, pl.ds(g * L, L)]
    sg = plsc.load_gather(snbr_ref, [idx])
    su = sself_ref[pl.ds(blk * G + g, L)][0]
    x = sg + su
    e = jnp.where(x >= 0.0, x, x * jnp.float32(0.01))
    m = jnp.max(e)
    ex = jnp.exp(e - m)
    z = jnp.sum(ex)
    alpha = ex / lax.broadcast_in_dim(z, (L,), ())
    acc = [jnp.zeros((L,), jnp.float32) for _ in range(DOUT // L)]
    for k in range(DEG):
        ak = alpha[k]
        row = g * DEG + k
        for j in range(DOUT // L):
            acc[j] = acc[j] + ak * rows_ref[row, pl.ds(j * L, L)]
    for j in range(DOUT // L):
        out_ref[g, pl.ds(j * L, L)] = acc[j]


def _sc_body(h_hbm, snbr_hbm, sself_hbm, nbrs_hbm, out_hbm,
             snbr_v, sself_v, nbrs_v, rows_v, out_v, gsem0, gsem1, osem0, osem1):
    wid = lax.axis_index("s") * 2 + lax.axis_index("c")
    base = wid * PW
    gsems = (gsem0, gsem1)
    osems = (osem0, osem1)
    pltpu.sync_copy(snbr_hbm, snbr_v)
    pltpu.sync_copy(sself_hbm.at[wid], sself_v)
    pltpu.sync_copy(nbrs_hbm.at[wid], nbrs_v)

    def start_gather(blk, buf):
        pltpu.make_async_copy(
            h_hbm.at[nbrs_v.at[blk]], rows_v.at[buf], gsems[buf]).start()

    def wait_gather(blk, buf):
        pltpu.make_async_copy(
            h_hbm.at[nbrs_v.at[blk]], rows_v.at[buf], gsems[buf]).wait()

    def start_out(blk, buf):
        pltpu.make_async_copy(
            out_v.at[buf], out_hbm.at[pl.ds(base + blk * G, G)], osems[buf]).start()

    def wait_out(blk, buf):
        pltpu.make_async_copy(
            out_v.at[buf], out_hbm.at[pl.ds(base + blk * G, G)], osems[buf]).wait()

    start_gather(0, 0)

    def pair_body(i2, carry):
        for b in range(2):
            blk = i2 * 2 + b

            @pl.when(blk + 1 < BLKS)
            def _():
                start_gather(blk + 1, 1 - b)

            wait_gather(blk, b)

            @pl.when(blk >= 2)
            def _():
                wait_out(blk - 2, b)

            def g_body(g, c2):
                _sc_node(g, blk, nbrs_v, sself_v, snbr_v, rows_v.at[b], out_v.at[b])
                return c2

            lax.fori_loop(0, G, g_body, 0)
            start_out(blk, b)
        return carry

    lax.fori_loop(0, BLKS // 2, pair_body, 0)
    wait_out(BLKS - 2, 0)
    wait_out(BLKS - 1, 1)


def _sc_stage(h, s_nbr, sself_w, nbrs_w):
    mesh = plsc.VectorSubcoreMesh(core_axis_name="c", subcore_axis_name="s")
    fn = pl.kernel(
        _sc_body,
        out_type=jax.ShapeDtypeStruct((NPAD, DOUT), jnp.float32),
        mesh=mesh,
        compiler_params=pltpu.CompilerParams(needs_layout_passes=False),
        scratch_types=[
            pltpu.VMEM((N,), jnp.float32),
            pltpu.VMEM((PW + L,), jnp.float32),
            pltpu.VMEM((BLKS, G * DEG), jnp.int32),
            pltpu.VMEM((2, G * DEG, DOUT), jnp.float32),
            pltpu.VMEM((2, G, DOUT), jnp.float32),
            pltpu.SemaphoreType.DMA,
            pltpu.SemaphoreType.DMA,
            pltpu.SemaphoreType.DMA,
            pltpu.SemaphoreType.DMA,
        ],
    )
    return fn(h, s_nbr, sself_w, nbrs_w)


@jax.jit
def _run(features, neighbors, W, b, a_w, a_b):
    A = jnp.zeros((DOUT, 128), jnp.float32)
    A = A.at[:, 0].set(a_w[:DOUT]).at[:, 1].set(a_w[DOUT:])
    c = jnp.zeros((1, 128), jnp.float32).at[0, 0].set(a_b)
    h, s2 = _tc_stage(features, W, b.reshape(1, DOUT), A, c)
    s_self = s2[:, 0]
    s_nbr = s2[:, 1]
    sself_w = jnp.pad(jnp.pad(s_self, (0, NPAD - N)).reshape(NW, PW), ((0, 0), (0, L)))
    nbrs_w = jnp.pad(neighbors, ((0, NPAD - N), (0, 0))).reshape(NW, BLKS, G * DEG)
    out = _sc_stage(h, s_nbr, sself_w, nbrs_w)
    return out[:N]


def kernel(features, nodes, neighbors, W, b, a_w, a_b):
    del nodes
    return _run(features, neighbors, W, b, a_w, a_b)
